# baseline (speedup 1.0000x reference)
import jax
import jax.numpy as jnp
from jax import lax
from jax.experimental import pallas as pl
from jax.experimental.pallas import tpu as pltpu

ROWS = 8192
COLS = 2048
BR = 1024
NB = ROWS // BR


def _body(pos_ref, img_ref, out_ref):
    pos = pos_ref[0, 0, :]
    cols = lax.broadcasted_iota(jnp.int32, (BR, COLS), 1)
    out_ref[:, :] = jnp.where(cols < pos[:, None], img_ref[:, :], 0.0)


@jax.jit
def _call(images, position):
    pos3 = position.reshape(NB, 1, BR)
    return pl.pallas_call(
        _body,
        grid=(NB,),
        in_specs=[
            pl.BlockSpec((1, 1, BR), lambda i: (i, 0, 0)),
            pl.BlockSpec((BR, COLS), lambda i: (i, 0)),
        ],
        out_specs=pl.BlockSpec((BR, COLS), lambda i: (i, 0)),
        out_shape=jax.ShapeDtypeStruct((ROWS, COLS), jnp.float32),
        compiler_params=pltpu.CompilerParams(
            dimension_semantics=("parallel",),
        ),
    )(pos3, images)


def kernel(images, position):
    return _call(images, position)
